# Initial kernel scaffold; baseline (speedup 1.0000x reference)
import jax
import jax.numpy as jnp
from jax.experimental import pallas as pl


def kernel(x, adj, w1, b1, w2, b2, wo, bo):
    raise NotImplementedError("write your pallas kernel here")


# speedup vs baseline: 10.4155x; 10.4155x over previous
import jax
import jax.numpy as jnp
from jax.experimental import pallas as pl
from jax.experimental.pallas import tpu as pltpu


def _round_up(v, m):
    return (v + m - 1) // m * m


def _stage1_kernel(adj_ref, xa_ref, w1a_ref, w2_ref, b2_ref,
                   h1_ref, h2lin_ref, acc_ref):
    k = pl.program_id(1)

    @pl.when(k == 0)
    def _init():
        acc_ref[...] = jnp.zeros_like(acc_ref)

    a16 = adj_ref[...].astype(jnp.bfloat16)
    acc_ref[...] += jnp.dot(a16, xa_ref[...],
                            preferred_element_type=jnp.float32)

    @pl.when(k == pl.num_programs(1) - 1)
    def _fin():
        ax = acc_ref[...].astype(jnp.bfloat16)
        h1 = jnp.maximum(
            jnp.dot(ax, w1a_ref[...], preferred_element_type=jnp.float32), 0.0)
        h1_bf = h1.astype(jnp.bfloat16)
        h1_ref[...] = h1_bf
        h2lin = jnp.dot(h1_bf, w2_ref[...],
                        preferred_element_type=jnp.float32) + b2_ref[...]
        h2lin_ref[...] = h2lin.astype(jnp.bfloat16)


def _stage2_kernel(adj_ref, h2lin_ref, h1_ref, wo_ref, bo_ref,
                   out_ref, acc_ref):
    k = pl.program_id(1)

    @pl.when(k == 0)
    def _init():
        acc_ref[...] = jnp.zeros_like(acc_ref)

    a16 = adj_ref[...].astype(jnp.bfloat16)
    acc_ref[...] += jnp.dot(a16, h2lin_ref[...],
                            preferred_element_type=jnp.float32)

    @pl.when(k == pl.num_programs(1) - 1)
    def _fin():
        h2 = jnp.maximum(acc_ref[...], 0.0)
        mixed = (h1_ref[...].astype(jnp.float32) + h2).astype(jnp.bfloat16)
        logits = jnp.dot(mixed, wo_ref[...],
                         preferred_element_type=jnp.float32)
        out_ref[...] = logits + bo_ref[...]


def _largest_tile(limit, size):
    t = limit
    while size % t:
        t //= 2
    return t


def kernel(x, adj, w1, b1, w2, b2, wo, bo):
    n, c = x.shape
    h = w1.shape[1]
    k_out = wo.shape[1]

    lane = 128
    c_aug = _round_up(c + 1, lane)
    h_pad = _round_up(h, lane)
    k_pad = _round_up(k_out, lane)
    n_pad = _round_up(n, lane)

    tile_m = _largest_tile(1024, n_pad)
    tile_k = _largest_tile(512, n_pad)

    bf = jnp.bfloat16

    x_aug = jnp.zeros((n_pad, c_aug), bf)
    x_aug = x_aug.at[:n, :c].set(x.astype(bf))
    x_aug = x_aug.at[:n, c].set(jnp.asarray(1.0, bf))

    if n_pad != n:
        adj_p = jnp.zeros((n_pad, n_pad), jnp.float32)
        adj_p = adj_p.at[:n, :n].set(adj.astype(jnp.float32))
    else:
        adj_p = adj.astype(jnp.float32)

    w1_aug = jnp.zeros((c_aug, h_pad), bf)
    w1_aug = w1_aug.at[:c, :h].set(w1.astype(bf))
    w1_aug = w1_aug.at[c, :h].set(b1.reshape(-1).astype(bf))

    w2_p = jnp.zeros((h_pad, h_pad), bf).at[:h, :h].set(w2.astype(bf))
    b2_p = jnp.zeros((1, h_pad), jnp.float32).at[0, :h].set(
        b2.reshape(-1).astype(jnp.float32))

    wo_p = jnp.zeros((h_pad, k_pad), bf).at[:h, :k_out].set(wo.astype(bf))
    bo_p = jnp.zeros((1, k_pad), jnp.float32).at[0, :k_out].set(
        bo.reshape(-1).astype(jnp.float32))

    grid = (n_pad // tile_m, n_pad // tile_k)
    cparams = pltpu.CompilerParams(
        dimension_semantics=("parallel", "arbitrary"),
        vmem_limit_bytes=64 * 1024 * 1024,
    )

    h1_p, h2lin_p = pl.pallas_call(
        _stage1_kernel,
        out_shape=(jax.ShapeDtypeStruct((n_pad, h_pad), bf),
                   jax.ShapeDtypeStruct((n_pad, h_pad), bf)),
        grid_spec=pltpu.PrefetchScalarGridSpec(
            num_scalar_prefetch=0,
            grid=grid,
            in_specs=[
                pl.BlockSpec((tile_m, tile_k), lambda i, k: (i, k)),
                pl.BlockSpec((tile_k, c_aug), lambda i, k: (k, 0)),
                pl.BlockSpec((c_aug, h_pad), lambda i, k: (0, 0)),
                pl.BlockSpec((h_pad, h_pad), lambda i, k: (0, 0)),
                pl.BlockSpec((1, h_pad), lambda i, k: (0, 0)),
            ],
            out_specs=(
                pl.BlockSpec((tile_m, h_pad), lambda i, k: (i, 0)),
                pl.BlockSpec((tile_m, h_pad), lambda i, k: (i, 0)),
            ),
            scratch_shapes=[pltpu.VMEM((tile_m, c_aug), jnp.float32)],
        ),
        compiler_params=cparams,
    )(adj_p, x_aug, w1_aug, w2_p, b2_p)

    logits_p = pl.pallas_call(
        _stage2_kernel,
        out_shape=jax.ShapeDtypeStruct((n_pad, k_pad), jnp.float32),
        grid_spec=pltpu.PrefetchScalarGridSpec(
            num_scalar_prefetch=0,
            grid=grid,
            in_specs=[
                pl.BlockSpec((tile_m, tile_k), lambda i, k: (i, k)),
                pl.BlockSpec((tile_k, h_pad), lambda i, k: (k, 0)),
                pl.BlockSpec((tile_m, h_pad), lambda i, k: (i, 0)),
                pl.BlockSpec((h_pad, k_pad), lambda i, k: (0, 0)),
                pl.BlockSpec((1, k_pad), lambda i, k: (0, 0)),
            ],
            out_specs=pl.BlockSpec((tile_m, k_pad), lambda i, k: (i, 0)),
            scratch_shapes=[pltpu.VMEM((tile_m, h_pad), jnp.float32)],
        ),
        compiler_params=cparams,
    )(adj_p, h2lin_p, h1_p, wo_p, bo_p)

    return logits_p[:n, :k_out]


# speedup vs baseline: 14.2059x; 1.3639x over previous
import jax
import jax.numpy as jnp
from jax.experimental import pallas as pl
from jax.experimental.pallas import tpu as pltpu


def _round_up(v, m):
    return (v + m - 1) // m * m


def _stage1_kernel(adj_ref, x_ref, w1_ref, b1_ref, w2_ref, b2_ref,
                   h1_ref, h2lin_ref, acc_ref, rs_ref):
    k = pl.program_id(1)

    @pl.when(k == 0)
    def _init():
        acc_ref[...] = jnp.zeros_like(acc_ref)
        rs_ref[...] = jnp.zeros_like(rs_ref)

    a = adj_ref[...]
    acc_ref[...] += jnp.dot(a, x_ref[...], preferred_element_type=jnp.float32)
    rs_ref[...] += jnp.sum(a, axis=1, keepdims=True)

    @pl.when(k == pl.num_programs(1) - 1)
    def _fin():
        pre1 = jnp.dot(acc_ref[...], w1_ref[...],
                       preferred_element_type=jnp.float32)
        pre1 += rs_ref[...] * b1_ref[...]
        h1 = jnp.maximum(pre1, 0.0)
        h1_ref[...] = h1
        h2lin_ref[...] = jnp.dot(h1, w2_ref[...],
                                 preferred_element_type=jnp.float32) + b2_ref[...]


def _stage2_kernel(adj_ref, h2lin_ref, h1_ref, wo_ref, bo_ref,
                   out_ref, acc_ref):
    k = pl.program_id(1)

    @pl.when(k == 0)
    def _init():
        acc_ref[...] = jnp.zeros_like(acc_ref)

    acc_ref[...] += jnp.dot(adj_ref[...], h2lin_ref[...],
                            preferred_element_type=jnp.float32)

    @pl.when(k == pl.num_programs(1) - 1)
    def _fin():
        h2 = jnp.maximum(acc_ref[...], 0.0)
        logits = jnp.dot(h1_ref[...] + h2, wo_ref[...],
                         preferred_element_type=jnp.float32)
        out_ref[...] = logits + bo_ref[...]


def _largest_tile(limit, size):
    t = limit
    while size % t:
        t //= 2
    return t


def kernel(x, adj, w1, b1, w2, b2, wo, bo):
    n, c = x.shape
    h = w1.shape[1]
    k_out = wo.shape[1]

    lane = 128
    c_pad = _round_up(c, lane)
    h_pad = _round_up(h, lane)
    k_pad = _round_up(k_out, lane)
    n_pad = _round_up(n, lane)

    tile_m = _largest_tile(1024, n_pad)
    tile_k = _largest_tile(1024, n_pad)

    f32 = jnp.float32

    def _pad2(a, r, cc):
        a = a.astype(f32)
        if a.shape == (r, cc):
            return a
        return jnp.zeros((r, cc), f32).at[:a.shape[0], :a.shape[1]].set(a)

    x_p = _pad2(x, n_pad, c_pad)
    adj_p = _pad2(adj, n_pad, n_pad)
    w1_p = _pad2(w1, c_pad, h_pad)
    b1_p = _pad2(b1.reshape(1, -1), 1, h_pad)
    w2_p = _pad2(w2, h_pad, h_pad)
    b2_p = _pad2(b2.reshape(1, -1), 1, h_pad)
    wo_p = _pad2(wo, h_pad, k_pad)
    bo_p = _pad2(bo.reshape(1, -1), 1, k_pad)

    grid = (n_pad // tile_m, n_pad // tile_k)
    cparams = pltpu.CompilerParams(
        dimension_semantics=("parallel", "arbitrary"),
        vmem_limit_bytes=64 * 1024 * 1024,
    )

    h1_p, h2lin_p = pl.pallas_call(
        _stage1_kernel,
        out_shape=(jax.ShapeDtypeStruct((n_pad, h_pad), f32),
                   jax.ShapeDtypeStruct((n_pad, h_pad), f32)),
        grid_spec=pltpu.PrefetchScalarGridSpec(
            num_scalar_prefetch=0,
            grid=grid,
            in_specs=[
                pl.BlockSpec((tile_m, tile_k), lambda i, k: (i, k)),
                pl.BlockSpec((tile_k, c_pad), lambda i, k: (k, 0)),
                pl.BlockSpec((c_pad, h_pad), lambda i, k: (0, 0)),
                pl.BlockSpec((1, h_pad), lambda i, k: (0, 0)),
                pl.BlockSpec((h_pad, h_pad), lambda i, k: (0, 0)),
                pl.BlockSpec((1, h_pad), lambda i, k: (0, 0)),
            ],
            out_specs=(
                pl.BlockSpec((tile_m, h_pad), lambda i, k: (i, 0)),
                pl.BlockSpec((tile_m, h_pad), lambda i, k: (i, 0)),
            ),
            scratch_shapes=[pltpu.VMEM((tile_m, c_pad), jnp.float32),
                            pltpu.VMEM((tile_m, 1), jnp.float32)],
        ),
        compiler_params=cparams,
    )(adj_p, x_p, w1_p, b1_p, w2_p, b2_p)

    logits_p = pl.pallas_call(
        _stage2_kernel,
        out_shape=jax.ShapeDtypeStruct((n_pad, k_pad), jnp.float32),
        grid_spec=pltpu.PrefetchScalarGridSpec(
            num_scalar_prefetch=0,
            grid=grid,
            in_specs=[
                pl.BlockSpec((tile_m, tile_k), lambda i, k: (i, k)),
                pl.BlockSpec((tile_k, h_pad), lambda i, k: (k, 0)),
                pl.BlockSpec((tile_m, h_pad), lambda i, k: (i, 0)),
                pl.BlockSpec((h_pad, k_pad), lambda i, k: (0, 0)),
                pl.BlockSpec((1, k_pad), lambda i, k: (0, 0)),
            ],
            out_specs=pl.BlockSpec((tile_m, k_pad), lambda i, k: (i, 0)),
            scratch_shapes=[pltpu.VMEM((tile_m, h_pad), jnp.float32)],
        ),
        compiler_params=cparams,
    )(adj_p, h2lin_p, h1_p, wo_p, bo_p)

    return logits_p[:n, :k_out]


# speedup vs baseline: 16.1142x; 1.1343x over previous
import jax
import jax.numpy as jnp
from jax.experimental import pallas as pl
from jax.experimental.pallas import tpu as pltpu


def _round_up(v, m):
    return (v + m - 1) // m * m


def _stage1_kernel(adj_ref, x_ref, w1_ref, b1_ref, w2_ref, b2_ref,
                   h1_ref, h2lin_ref, acc_ref, rs_ref):
    k = pl.program_id(1)

    @pl.when(k == 0)
    def _init():
        acc_ref[...] = jnp.zeros_like(acc_ref)
        rs_ref[...] = jnp.zeros_like(rs_ref)

    a = adj_ref[...]
    acc_ref[...] += jnp.dot(a, x_ref[...], preferred_element_type=jnp.float32)
    rs_ref[...] += jnp.sum(a, axis=1, keepdims=True)

    @pl.when(k == pl.num_programs(1) - 1)
    def _fin():
        pre1 = jnp.dot(acc_ref[...], w1_ref[...],
                       preferred_element_type=jnp.float32)
        pre1 += rs_ref[...] * b1_ref[...]
        h1 = jnp.maximum(pre1, 0.0)
        h1_ref[...] = h1.astype(h1_ref.dtype)
        h2lin = jnp.dot(h1, w2_ref[...],
                        preferred_element_type=jnp.float32) + b2_ref[...]
        h2lin_ref[...] = h2lin.astype(h2lin_ref.dtype)


def _stage2_kernel(adj_ref, h2lin_ref, h1_ref, wo_ref, bo_ref,
                   out_ref, acc_ref):
    k = pl.program_id(1)

    @pl.when(k == 0)
    def _init():
        acc_ref[...] = jnp.zeros_like(acc_ref)

    acc_ref[...] += jnp.dot(adj_ref[...], h2lin_ref[...].astype(jnp.float32),
                            preferred_element_type=jnp.float32)

    @pl.when(k == pl.num_programs(1) - 1)
    def _fin():
        h2 = jnp.maximum(acc_ref[...], 0.0)
        logits = jnp.dot(h1_ref[...].astype(jnp.float32) + h2, wo_ref[...],
                         preferred_element_type=jnp.float32)
        out_ref[...] = logits + bo_ref[...]


def _largest_tile(limit, size):
    t = limit
    while size % t:
        t //= 2
    return t


def kernel(x, adj, w1, b1, w2, b2, wo, bo):
    n, c = x.shape
    h = w1.shape[1]
    k_out = wo.shape[1]

    lane = 128
    c_pad = _round_up(c, lane)
    h_pad = _round_up(h, lane)
    k_pad = _round_up(k_out, lane)
    n_pad = _round_up(n, lane)

    tile_m = _largest_tile(2048, n_pad)
    tile_k = _largest_tile(512, n_pad)

    f32 = jnp.float32

    def _pad2(a, r, cc):
        a = a.astype(f32)
        if a.shape == (r, cc):
            return a
        return jnp.zeros((r, cc), f32).at[:a.shape[0], :a.shape[1]].set(a)

    x_p = _pad2(x, n_pad, c_pad)
    adj_p = _pad2(adj, n_pad, n_pad)
    w1_p = _pad2(w1, c_pad, h_pad)
    b1_p = _pad2(b1.reshape(1, -1), 1, h_pad)
    w2_p = _pad2(w2, h_pad, h_pad)
    b2_p = _pad2(b2.reshape(1, -1), 1, h_pad)
    wo_p = _pad2(wo, h_pad, k_pad)
    bo_p = _pad2(bo.reshape(1, -1), 1, k_pad)

    grid = (n_pad // tile_m, n_pad // tile_k)
    cparams = pltpu.CompilerParams(
        dimension_semantics=("parallel", "arbitrary"),
        vmem_limit_bytes=64 * 1024 * 1024,
    )

    h1_p, h2lin_p = pl.pallas_call(
        _stage1_kernel,
        out_shape=(jax.ShapeDtypeStruct((n_pad, h_pad), jnp.bfloat16),
                   jax.ShapeDtypeStruct((n_pad, h_pad), jnp.bfloat16)),
        grid_spec=pltpu.PrefetchScalarGridSpec(
            num_scalar_prefetch=0,
            grid=grid,
            in_specs=[
                pl.BlockSpec((tile_m, tile_k), lambda i, k: (i, k)),
                pl.BlockSpec((tile_k, c_pad), lambda i, k: (k, 0)),
                pl.BlockSpec((c_pad, h_pad), lambda i, k: (0, 0)),
                pl.BlockSpec((1, h_pad), lambda i, k: (0, 0)),
                pl.BlockSpec((h_pad, h_pad), lambda i, k: (0, 0)),
                pl.BlockSpec((1, h_pad), lambda i, k: (0, 0)),
            ],
            out_specs=(
                pl.BlockSpec((tile_m, h_pad), lambda i, k: (i, 0)),
                pl.BlockSpec((tile_m, h_pad), lambda i, k: (i, 0)),
            ),
            scratch_shapes=[pltpu.VMEM((tile_m, c_pad), jnp.float32),
                            pltpu.VMEM((tile_m, 1), jnp.float32)],
        ),
        compiler_params=cparams,
    )(adj_p, x_p, w1_p, b1_p, w2_p, b2_p)

    logits_p = pl.pallas_call(
        _stage2_kernel,
        out_shape=jax.ShapeDtypeStruct((n_pad, k_pad), jnp.float32),
        grid_spec=pltpu.PrefetchScalarGridSpec(
            num_scalar_prefetch=0,
            grid=grid,
            in_specs=[
                pl.BlockSpec((tile_m, tile_k), lambda i, k: (i, k)),
                pl.BlockSpec((tile_k, h_pad), lambda i, k: (k, 0)),
                pl.BlockSpec((tile_m, h_pad), lambda i, k: (i, 0)),
                pl.BlockSpec((h_pad, k_pad), lambda i, k: (0, 0)),
                pl.BlockSpec((1, k_pad), lambda i, k: (0, 0)),
            ],
            out_specs=pl.BlockSpec((tile_m, k_pad), lambda i, k: (i, 0)),
            scratch_shapes=[pltpu.VMEM((tile_m, h_pad), jnp.float32)],
        ),
        compiler_params=cparams,
    )(adj_p, h2lin_p, h1_p, wo_p, bo_p)

    return logits_p[:n, :k_out]
